# baseline (speedup 1.0000x reference)
import functools

import jax
import jax.numpy as jnp
from jax import lax
from jax.experimental import pallas as pl
from jax.experimental.pallas import tpu as pltpu
from jax.experimental.pallas import tpu_sc as plsc


def _combine_tables(depth_embed, width_embed):
    VD, D = depth_embed.shape
    VW, _ = width_embed.shape

    def body(d_ref, w_ref, o_ref):
        i = pl.program_id(0)
        w = w_ref[...]
        r0 = d_ref[pl.ds(2 * i, 1), :]
        r1 = d_ref[pl.ds(2 * i + 1, 1), :]
        o_ref[0] = jnp.concatenate([r0 + w, r1 + w], axis=0)

    out3 = pl.pallas_call(
        body,
        grid=(VD // 2,),
        in_specs=[
            pl.BlockSpec((VD, D), lambda i: (0, 0)),
            pl.BlockSpec((VW, D), lambda i: (0, 0)),
        ],
        out_specs=pl.BlockSpec((1, 2 * VW, D), lambda i: (i, 0, 0)),
        out_shape=jax.ShapeDtypeStruct((VD // 2, 2 * VW, D), jnp.float32),
    )(depth_embed, width_embed)
    return out3.reshape(VD * VW, D)


@functools.partial(jax.jit, static_argnums=(3, 4, 5))
def _sc_gather(d_idx, w_idx, comb, d_model, vd_max, vw_max):
    seq = d_idx.shape[0]
    info = plsc.get_sparse_core_info()
    NC, NS = info.num_cores, info.num_subcores
    NW = NC * NS
    bpw = seq // NW
    C = 32
    nch = bpw // C
    per = C // 16

    mesh = plsc.VectorSubcoreMesh(core_axis_name="c", subcore_axis_name="s")

    @functools.partial(
        pl.kernel,
        mesh=mesh,
        out_type=jax.ShapeDtypeStruct((seq, 1, d_model), jnp.float32),
        scratch_types=[
            pltpu.VMEM((bpw,), jnp.int32),
            pltpu.VMEM((bpw,), jnp.int32),
            pltpu.VMEM((nch, C), jnp.int32),
            pltpu.VMEM((C, d_model), jnp.float32),
            pltpu.VMEM((C, d_model), jnp.float32),
            pltpu.SemaphoreType.DMA,
            pltpu.SemaphoreType.DMA,
            pltpu.SemaphoreType.DMA,
            pltpu.SemaphoreType.DMA,
        ],
    )
    def k(d_hbm, w_hbm, comb_hbm, out_hbm, dv, wv, cidx,
          buf0, buf1, gsem0, gsem1, ssem0, ssem1):
        wid = lax.axis_index("s") * NC + lax.axis_index("c")
        base = wid * bpw
        bufs = (buf0, buf1)
        gsems = (gsem0, gsem1)
        ssems = (ssem0, ssem1)
        pltpu.sync_copy(d_hbm.at[pl.ds(base, bpw)], dv)
        pltpu.sync_copy(w_hbm.at[pl.ds(base, bpw)], wv)
        for kk in range(bpw // 16):
            d = dv[pl.ds(kk * 16, 16)]
            w = wv[pl.ds(kk * 16, 16)]
            d = jnp.minimum(jnp.maximum(d, 0), vd_max)
            w = jnp.minimum(jnp.maximum(w, 0), vw_max)
            cidx[kk // per, pl.ds((kk % per) * 16, 16)] = d * (vw_max + 1) + w

        def gather(j, b):
            return pltpu.async_copy(comb_hbm.at[cidx.at[j]], bufs[b], gsems[b])

        def scatter(j, b):
            return pltpu.async_copy(
                bufs[b], out_hbm.at[pl.ds(base + j * C, C), 0], ssems[b])

        gd = [None, None]
        sd = [None, None]
        gd[0] = gather(0, 0)
        for j in range(nch):
            cur = j % 2
            oth = 1 - cur
            gd[cur].wait()
            if j + 1 < nch:
                if j >= 1:
                    sd[oth].wait()
                gd[oth] = gather(j + 1, oth)
            sd[cur] = scatter(j, cur)
        sd[0].wait()
        sd[1].wait()

    return k(d_idx, w_idx, comb)


def kernel(depth_indices, width_indices, depth_embed, width_embed):
    seq = depth_indices.shape[0]
    D = depth_embed.shape[1]
    comb = _combine_tables(depth_embed, width_embed)
    d = depth_indices.reshape(seq).astype(jnp.int32)
    w = width_indices.reshape(seq).astype(jnp.int32)
    return _sc_gather(d, w, comb, D,
                      depth_embed.shape[0] - 1, width_embed.shape[0] - 1)


# speedup vs baseline: 1.1022x; 1.1022x over previous
import functools

import jax
import jax.numpy as jnp
from jax import lax
from jax.experimental import pallas as pl
from jax.experimental.pallas import tpu as pltpu
from jax.experimental.pallas import tpu_sc as plsc


def _combine_tables(depth_embed, width_embed):
    VD, D = depth_embed.shape
    VW, _ = width_embed.shape

    WP = -(-VW // 8) * 8
    w_pad = jnp.concatenate(
        [width_embed, jnp.zeros((WP - VW, D), jnp.float32)], axis=0)

    def body(d_ref, w_ref, o_ref):
        d = d_ref[...]
        w = w_ref[...]
        o_ref[...] = d[:, None, :] + w[None, :, :]

    out3 = pl.pallas_call(
        body,
        out_shape=jax.ShapeDtypeStruct((VD, WP, D), jnp.float32),
    )(depth_embed, w_pad)
    return out3.reshape(VD * WP, D)


@functools.partial(jax.jit, static_argnums=(3, 4, 5))
def _sc_gather(d_idx, w_idx, comb, d_model, vd_max, vw_max):
    seq = d_idx.shape[0]
    info = plsc.get_sparse_core_info()
    NC, NS = info.num_cores, info.num_subcores
    NW = NC * NS
    bpw = seq // NW
    C = 32
    nch = bpw // C
    per = C // 16
    STR = -(-(vw_max + 1) // 8) * 8

    mesh = plsc.VectorSubcoreMesh(core_axis_name="c", subcore_axis_name="s")

    @functools.partial(
        pl.kernel,
        mesh=mesh,
        out_type=jax.ShapeDtypeStruct((seq, 1, d_model), jnp.float32),
        scratch_types=[
            pltpu.VMEM((bpw,), jnp.int32),
            pltpu.VMEM((bpw,), jnp.int32),
            pltpu.VMEM((nch, C), jnp.int32),
            pltpu.VMEM((C, d_model), jnp.float32),
            pltpu.VMEM((C, d_model), jnp.float32),
            pltpu.SemaphoreType.DMA,
            pltpu.SemaphoreType.DMA,
            pltpu.SemaphoreType.DMA,
            pltpu.SemaphoreType.DMA,
        ],
    )
    def k(d_hbm, w_hbm, comb_hbm, out_hbm, dv, wv, cidx,
          buf0, buf1, gsem0, gsem1, ssem0, ssem1):
        wid = lax.axis_index("s") * NC + lax.axis_index("c")
        base = wid * bpw
        bufs = (buf0, buf1)
        gsems = (gsem0, gsem1)
        ssems = (ssem0, ssem1)
        pltpu.sync_copy(d_hbm.at[pl.ds(base, bpw)], dv)
        pltpu.sync_copy(w_hbm.at[pl.ds(base, bpw)], wv)
        for kk in range(bpw // 16):
            d = dv[pl.ds(kk * 16, 16)]
            w = wv[pl.ds(kk * 16, 16)]
            d = jnp.minimum(jnp.maximum(d, 0), vd_max)
            w = jnp.minimum(jnp.maximum(w, 0), vw_max)
            cidx[kk // per, pl.ds((kk % per) * 16, 16)] = d * STR + w

        def gather(j, b):
            return pltpu.async_copy(comb_hbm.at[cidx.at[j]], bufs[b], gsems[b])

        def scatter(j, b):
            return pltpu.async_copy(
                bufs[b], out_hbm.at[pl.ds(base + j * C, C), 0], ssems[b])

        gd = [None, None]
        sd = [None, None]
        gd[0] = gather(0, 0)
        for j in range(nch):
            cur = j % 2
            oth = 1 - cur
            gd[cur].wait()
            if j + 1 < nch:
                if j >= 1:
                    sd[oth].wait()
                gd[oth] = gather(j + 1, oth)
            sd[cur] = scatter(j, cur)
        sd[0].wait()
        sd[1].wait()

    return k(d_idx, w_idx, comb)


def kernel(depth_indices, width_indices, depth_embed, width_embed):
    seq = depth_indices.shape[0]
    D = depth_embed.shape[1]
    comb = _combine_tables(depth_embed, width_embed)
    d = depth_indices.reshape(seq).astype(jnp.int32)
    w = width_indices.reshape(seq).astype(jnp.int32)
    return _sc_gather(d, w, comb, D,
                      depth_embed.shape[0] - 1, width_embed.shape[0] - 1)


# speedup vs baseline: 1.1562x; 1.0489x over previous
import functools

import jax
import jax.numpy as jnp
from jax import lax
from jax.experimental import pallas as pl
from jax.experimental.pallas import tpu as pltpu
from jax.experimental.pallas import tpu_sc as plsc


def _combine_tables(depth_embed, width_embed):
    VD, D = depth_embed.shape
    VW, _ = width_embed.shape

    WP = -(-VW // 8) * 8
    w_pad = jnp.concatenate(
        [width_embed, jnp.zeros((WP - VW, D), jnp.float32)], axis=0)

    def body(d_ref, w_ref, o_ref):
        d = d_ref[...]
        w = w_ref[...]
        o_ref[...] = d[:, None, :] + w[None, :, :]

    out3 = pl.pallas_call(
        body,
        out_shape=jax.ShapeDtypeStruct((VD, WP, D), jnp.float32),
    )(depth_embed, w_pad)
    return out3.reshape(VD * WP, D)


@functools.partial(jax.jit, static_argnums=(3, 4, 5))
def _sc_gather(d_idx, w_idx, comb, d_model, vd_max, vw_max):
    seq = d_idx.shape[0]
    info = plsc.get_sparse_core_info()
    NC, NS = info.num_cores, info.num_subcores
    NW = NC * NS
    bpw = seq // NW
    C = 16
    NB = 4
    PRIME = 2
    nch = bpw // C
    per = max(C // 16, 1)
    STR = -(-(vw_max + 1) // 8) * 8

    mesh = plsc.VectorSubcoreMesh(core_axis_name="c", subcore_axis_name="s")

    @functools.partial(
        pl.kernel,
        mesh=mesh,
        out_type=jax.ShapeDtypeStruct((seq, 1, d_model), jnp.float32),
        scratch_types=[
            pltpu.VMEM((bpw,), jnp.int32),
            pltpu.VMEM((bpw,), jnp.int32),
            pltpu.VMEM((nch, C), jnp.int32),
            pltpu.VMEM((C, d_model), jnp.float32),
            pltpu.VMEM((C, d_model), jnp.float32),
            pltpu.VMEM((C, d_model), jnp.float32),
            pltpu.VMEM((C, d_model), jnp.float32),
            pltpu.SemaphoreType.DMA,
            pltpu.SemaphoreType.DMA,
            pltpu.SemaphoreType.DMA,
            pltpu.SemaphoreType.DMA,
            pltpu.SemaphoreType.DMA,
            pltpu.SemaphoreType.DMA,
            pltpu.SemaphoreType.DMA,
            pltpu.SemaphoreType.DMA,
        ],
    )
    def k(d_hbm, w_hbm, comb_hbm, out_hbm, dv, wv, cidx,
          buf0, buf1, buf2, buf3,
          gsem0, gsem1, gsem2, gsem3, ssem0, ssem1, ssem2, ssem3):
        wid = lax.axis_index("s") * NC + lax.axis_index("c")
        base = wid * bpw
        bufs = (buf0, buf1, buf2, buf3)
        gsems = (gsem0, gsem1, gsem2, gsem3)
        ssems = (ssem0, ssem1, ssem2, ssem3)
        pltpu.sync_copy(d_hbm.at[pl.ds(base, bpw)], dv)
        pltpu.sync_copy(w_hbm.at[pl.ds(base, bpw)], wv)
        for kk in range(bpw // 16):
            d = dv[pl.ds(kk * 16, 16)]
            w = wv[pl.ds(kk * 16, 16)]
            d = jnp.minimum(jnp.maximum(d, 0), vd_max)
            w = jnp.minimum(jnp.maximum(w, 0), vw_max)
            cidx[kk // per, pl.ds((kk % per) * 16, 16)] = d * STR + w

        def gather(j, b):
            return pltpu.async_copy(comb_hbm.at[cidx.at[j]], bufs[b], gsems[b])

        def scatter(j, b):
            return pltpu.async_copy(
                bufs[b], out_hbm.at[pl.ds(base + j * C, C), 0], ssems[b])

        gd = [None] * NB
        sd = [None] * NB
        for j in range(min(PRIME, nch)):
            gd[j % NB] = gather(j, j % NB)
        for j in range(nch):
            b = j % NB
            gd[b].wait()
            jn = j + PRIME
            if jn < nch:
                bn = jn % NB
                if sd[bn] is not None:
                    sd[bn].wait()
                gd[bn] = gather(jn, bn)
            sd[b] = scatter(j, b)
        for b in range(NB):
            if sd[b] is not None:
                sd[b].wait()

    return k(d_idx, w_idx, comb)


def kernel(depth_indices, width_indices, depth_embed, width_embed):
    seq = depth_indices.shape[0]
    D = depth_embed.shape[1]
    comb = _combine_tables(depth_embed, width_embed)
    d = depth_indices.reshape(seq).astype(jnp.int32)
    w = width_indices.reshape(seq).astype(jnp.int32)
    return _sc_gather(d, w, comb, D,
                      depth_embed.shape[0] - 1, width_embed.shape[0] - 1)


# speedup vs baseline: 1.1562x; 1.0000x over previous
import functools

import jax
import jax.numpy as jnp
from jax import lax
from jax.experimental import pallas as pl
from jax.experimental.pallas import tpu as pltpu
from jax.experimental.pallas import tpu_sc as plsc


def _combine_tables(depth_embed, width_embed):
    VD, D = depth_embed.shape
    VW, _ = width_embed.shape

    WP = -(-VW // 8) * 8
    w_pad = jnp.concatenate(
        [width_embed, jnp.zeros((WP - VW, D), jnp.float32)], axis=0)

    def body(d_ref, w_ref, o_ref):
        d = d_ref[...]
        w = w_ref[...]
        o_ref[...] = d[:, None, :] + w[None, :, :]

    out3 = pl.pallas_call(
        body,
        out_shape=jax.ShapeDtypeStruct((VD, WP, D), jnp.float32),
    )(depth_embed, w_pad)
    return out3.reshape(VD * WP, D)


@functools.partial(jax.jit, static_argnums=(3, 4, 5))
def _sc_gather(d_idx, w_idx, comb, d_model, vd_max, vw_max):
    seq = d_idx.shape[0]
    info = plsc.get_sparse_core_info()
    NC, NS = info.num_cores, info.num_subcores
    NW = NC * NS
    bpw = seq // NW
    C = 16
    NB = 6
    PRIME = 3
    nch = bpw // C
    per = max(C // 16, 1)
    STR = -(-(vw_max + 1) // 8) * 8

    mesh = plsc.VectorSubcoreMesh(core_axis_name="c", subcore_axis_name="s")

    @functools.partial(
        pl.kernel,
        mesh=mesh,
        out_type=jax.ShapeDtypeStruct((seq, 1, d_model), jnp.float32),
        scratch_types=[
            pltpu.VMEM((bpw,), jnp.int32),
            pltpu.VMEM((bpw,), jnp.int32),
            pltpu.VMEM((nch, C), jnp.int32),
        ] + [pltpu.VMEM((C, d_model), jnp.float32)] * 6
          + [pltpu.SemaphoreType.DMA] * 12,
    )
    def k(d_hbm, w_hbm, comb_hbm, out_hbm, dv, wv, cidx, *rest):
        bufs = rest[:6]
        gsems = rest[6:12]
        ssems = rest[12:18]
        wid = lax.axis_index("s") * NC + lax.axis_index("c")
        base = wid * bpw
        pltpu.sync_copy(d_hbm.at[pl.ds(base, bpw)], dv)
        pltpu.sync_copy(w_hbm.at[pl.ds(base, bpw)], wv)
        for kk in range(bpw // 16):
            d = dv[pl.ds(kk * 16, 16)]
            w = wv[pl.ds(kk * 16, 16)]
            d = jnp.minimum(jnp.maximum(d, 0), vd_max)
            w = jnp.minimum(jnp.maximum(w, 0), vw_max)
            cidx[kk // per, pl.ds((kk % per) * 16, 16)] = d * STR + w

        def gather(j, b):
            return pltpu.async_copy(comb_hbm.at[cidx.at[j]], bufs[b], gsems[b])

        def scatter(j, b):
            return pltpu.async_copy(
                bufs[b], out_hbm.at[pl.ds(base + j * C, C), 0], ssems[b])

        gd = [None] * NB
        sd = [None] * NB
        for j in range(min(PRIME, nch)):
            gd[j % NB] = gather(j, j % NB)
        for j in range(nch):
            b = j % NB
            gd[b].wait()
            jn = j + PRIME
            if jn < nch:
                bn = jn % NB
                if sd[bn] is not None:
                    sd[bn].wait()
                gd[bn] = gather(jn, bn)
            sd[b] = scatter(j, b)
        for b in range(NB):
            if sd[b] is not None:
                sd[b].wait()

    return k(d_idx, w_idx, comb)


def kernel(depth_indices, width_indices, depth_embed, width_embed):
    seq = depth_indices.shape[0]
    D = depth_embed.shape[1]
    comb = _combine_tables(depth_embed, width_embed)
    d = depth_indices.reshape(seq).astype(jnp.int32)
    w = width_indices.reshape(seq).astype(jnp.int32)
    return _sc_gather(d, w, comb, D,
                      depth_embed.shape[0] - 1, width_embed.shape[0] - 1)


# speedup vs baseline: 1.1795x; 1.0201x over previous
import functools

import jax
import jax.numpy as jnp
from jax import lax
from jax.experimental import pallas as pl
from jax.experimental.pallas import tpu as pltpu
from jax.experimental.pallas import tpu_sc as plsc


def _combine_tables(depth_embed, width_embed):
    VD, D = depth_embed.shape
    VW, _ = width_embed.shape

    WP = -(-VW // 8) * 8

    def body(d_ref, w_ref, o_ref):
        d = d_ref[...]
        w = w_ref[...]
        if WP > VW:
            w = jnp.concatenate([w, w[: WP - VW]], axis=0)
        o_ref[...] = d[:, None, :] + w[None, :, :]

    out3 = pl.pallas_call(
        body,
        out_shape=jax.ShapeDtypeStruct((VD, WP, D), jnp.float32),
    )(depth_embed, width_embed)
    return out3.reshape(VD * WP, D)


@functools.partial(jax.jit, static_argnums=(3, 4, 5))
def _sc_gather(d_idx, w_idx, comb, d_model, vd_max, vw_max):
    seq = d_idx.shape[0]
    info = plsc.get_sparse_core_info()
    NC, NS = info.num_cores, info.num_subcores
    NW = NC * NS
    bpw = seq // NW
    C = 16
    NB = 6
    PRIME = 3
    nch = bpw // C
    per = max(C // 16, 1)
    STR = -(-(vw_max + 1) // 8) * 8

    mesh = plsc.VectorSubcoreMesh(core_axis_name="c", subcore_axis_name="s")

    @functools.partial(
        pl.kernel,
        mesh=mesh,
        out_type=jax.ShapeDtypeStruct((seq, 1, d_model), jnp.float32),
        scratch_types=[
            pltpu.VMEM((bpw,), jnp.int32),
            pltpu.VMEM((bpw,), jnp.int32),
            pltpu.VMEM((nch, C), jnp.int32),
        ] + [pltpu.VMEM((C, d_model), jnp.float32)] * 6
          + [pltpu.SemaphoreType.DMA] * 12,
    )
    def k(d_hbm, w_hbm, comb_hbm, out_hbm, dv, wv, cidx, *rest):
        bufs = rest[:6]
        gsems = rest[6:12]
        ssems = rest[12:18]
        wid = lax.axis_index("s") * NC + lax.axis_index("c")
        base = wid * bpw
        pltpu.sync_copy(d_hbm.at[pl.ds(base, bpw)], dv)
        pltpu.sync_copy(w_hbm.at[pl.ds(base, bpw)], wv)
        for kk in range(bpw // 16):
            d = dv[pl.ds(kk * 16, 16)]
            w = wv[pl.ds(kk * 16, 16)]
            d = jnp.minimum(jnp.maximum(d, 0), vd_max)
            w = jnp.minimum(jnp.maximum(w, 0), vw_max)
            cidx[kk // per, pl.ds((kk % per) * 16, 16)] = d * STR + w

        def gather(j, b):
            return pltpu.async_copy(comb_hbm.at[cidx.at[j]], bufs[b], gsems[b])

        def scatter(j, b):
            return pltpu.async_copy(
                bufs[b], out_hbm.at[pl.ds(base + j * C, C), 0], ssems[b])

        gd = [None] * NB
        sd = [None] * NB
        for j in range(min(PRIME, nch)):
            gd[j % NB] = gather(j, j % NB)
        for j in range(nch):
            b = j % NB
            gd[b].wait()
            jn = j + PRIME
            if jn < nch:
                bn = jn % NB
                if sd[bn] is not None:
                    sd[bn].wait()
                gd[bn] = gather(jn, bn)
            sd[b] = scatter(j, b)
        for b in range(NB):
            if sd[b] is not None:
                sd[b].wait()

    return k(d_idx, w_idx, comb)


def kernel(depth_indices, width_indices, depth_embed, width_embed):
    seq = depth_indices.shape[0]
    D = depth_embed.shape[1]
    comb = _combine_tables(depth_embed, width_embed)
    d = depth_indices.reshape(seq).astype(jnp.int32)
    w = width_indices.reshape(seq).astype(jnp.int32)
    return _sc_gather(d, w, comb, D,
                      depth_embed.shape[0] - 1, width_embed.shape[0] - 1)


# speedup vs baseline: 1.1963x; 1.0143x over previous
import functools

import jax
import jax.numpy as jnp
from jax import lax
from jax.experimental import pallas as pl
from jax.experimental.pallas import tpu as pltpu
from jax.experimental.pallas import tpu_sc as plsc


def _combine_tables(depth_embed, width_embed):
    VD, D = depth_embed.shape
    VW, _ = width_embed.shape

    WP = -(-VW // 8) * 8

    def body(d_ref, w_ref, o_ref):
        d = d_ref[...]
        w = w_ref[...]
        if WP > VW:
            w = jnp.concatenate([w, w[: WP - VW]], axis=0)
        o_ref[...] = d[:, None, :] + w[None, :, :]

    out3 = pl.pallas_call(
        body,
        out_shape=jax.ShapeDtypeStruct((VD, WP, D), jnp.float32),
    )(depth_embed, width_embed)
    return out3.reshape(VD * WP, D)


@functools.partial(jax.jit, static_argnums=(3, 4, 5))
def _sc_gather(d_idx, w_idx, comb, d_model, vd_max, vw_max):
    seq = d_idx.shape[0]
    info = plsc.get_sparse_core_info()
    NC, NS = info.num_cores, info.num_subcores
    NW = NC * NS
    bpw = seq // NW
    C = 16
    NB = 6
    PRIME = 3
    nch = bpw // C
    per = max(C // 16, 1)
    STR = -(-(vw_max + 1) // 8) * 8

    mesh = plsc.VectorSubcoreMesh(core_axis_name="c", subcore_axis_name="s")

    @functools.partial(
        pl.kernel,
        mesh=mesh,
        out_type=jax.ShapeDtypeStruct((seq, 1, d_model), jnp.float32),
        scratch_types=[
            pltpu.VMEM((bpw,), jnp.int32),
            pltpu.VMEM((bpw,), jnp.int32),
            pltpu.VMEM((nch, C), jnp.int32),
        ] + [pltpu.VMEM((C, d_model), jnp.float32)] * 6
          + [pltpu.SemaphoreType.DMA] * 14,
    )
    def k(d_hbm, w_hbm, comb_hbm, out_hbm, dv, wv, cidx, *rest):
        bufs = rest[:6]
        gsems = rest[6:12]
        ssems = rest[12:18]
        isems = rest[18:20]
        wid = lax.axis_index("s") * NC + lax.axis_index("c")
        base = wid * bpw
        dcp = pltpu.async_copy(d_hbm.at[pl.ds(base, bpw)], dv, isems[0])
        wcp = pltpu.async_copy(w_hbm.at[pl.ds(base, bpw)], wv, isems[1])
        dcp.wait()
        wcp.wait()

        def make_idx(j):
            for kk in range(j * per, (j + 1) * per):
                d = dv[pl.ds(kk * 16, 16)]
                w = wv[pl.ds(kk * 16, 16)]
                d = jnp.minimum(jnp.maximum(d, 0), vd_max)
                w = jnp.minimum(jnp.maximum(w, 0), vw_max)
                cidx[kk // per, pl.ds((kk % per) * 16, 16)] = d * STR + w

        def gather(j, b):
            return pltpu.async_copy(comb_hbm.at[cidx.at[j]], bufs[b], gsems[b])

        def scatter(j, b):
            return pltpu.async_copy(
                bufs[b], out_hbm.at[pl.ds(base + j * C, C), 0], ssems[b])

        gd = [None] * NB
        sd = [None] * NB
        for j in range(min(PRIME, nch)):
            make_idx(j)
            gd[j % NB] = gather(j, j % NB)
        for j in range(min(PRIME, nch), nch):
            make_idx(j)
        for j in range(nch):
            b = j % NB
            gd[b].wait()
            jn = j + PRIME
            if jn < nch:
                bn = jn % NB
                if sd[bn] is not None:
                    sd[bn].wait()
                gd[bn] = gather(jn, bn)
            sd[b] = scatter(j, b)
        for b in range(NB):
            if sd[b] is not None:
                sd[b].wait()

    return k(d_idx, w_idx, comb)


def kernel(depth_indices, width_indices, depth_embed, width_embed):
    seq = depth_indices.shape[0]
    D = depth_embed.shape[1]
    comb = _combine_tables(depth_embed, width_embed)
    d = depth_indices.reshape(seq).astype(jnp.int32)
    w = width_indices.reshape(seq).astype(jnp.int32)
    return _sc_gather(d, w, comb, D,
                      depth_embed.shape[0] - 1, width_embed.shape[0] - 1)


# speedup vs baseline: 1.2065x; 1.0085x over previous
import functools

import jax
import jax.numpy as jnp
from jax import lax
from jax.experimental import pallas as pl
from jax.experimental.pallas import tpu as pltpu
from jax.experimental.pallas import tpu_sc as plsc


def _combine_tables(depth_embed, width_embed):
    VD, D = depth_embed.shape
    VW, _ = width_embed.shape

    WP = -(-VW // 8) * 8

    def body(d_ref, w_ref, o_ref):
        d = d_ref[...]
        w = w_ref[...]
        if WP > VW:
            w = jnp.concatenate([w, w[: WP - VW]], axis=0)
        o_ref[...] = d[:, None, :] + w[None, :, :]

    out3 = pl.pallas_call(
        body,
        out_shape=jax.ShapeDtypeStruct((VD, WP, D), jnp.float32),
    )(depth_embed, width_embed)
    return out3.reshape(VD * WP, D)


@functools.partial(jax.jit, static_argnums=(3, 4, 5))
def _sc_gather(d_idx, w_idx, comb, d_model, vd_max, vw_max):
    seq = d_idx.shape[0]
    info = plsc.get_sparse_core_info()
    NC, NS = info.num_cores, info.num_subcores
    NW = NC * NS
    bpw = seq // NW
    C = 32
    NB = 3
    PRIME = 2
    nch = bpw // C
    per = max(C // 16, 1)
    STR = -(-(vw_max + 1) // 8) * 8

    mesh = plsc.VectorSubcoreMesh(core_axis_name="c", subcore_axis_name="s")

    @functools.partial(
        pl.kernel,
        mesh=mesh,
        out_type=jax.ShapeDtypeStruct((seq, 1, d_model), jnp.float32),
        scratch_types=[
            pltpu.VMEM((bpw,), jnp.int32),
            pltpu.VMEM((bpw,), jnp.int32),
            pltpu.VMEM((nch, C), jnp.int32),
        ] + [pltpu.VMEM((C, d_model), jnp.float32)] * 3
          + [pltpu.SemaphoreType.DMA] * 8,
    )
    def k(d_hbm, w_hbm, comb_hbm, out_hbm, dv, wv, cidx, *rest):
        bufs = rest[:3]
        gsems = rest[3:6]
        ssems = rest[6:9]
        isems = rest[9:11]
        wid = lax.axis_index("s") * NC + lax.axis_index("c")
        base = wid * bpw
        dcp = pltpu.async_copy(d_hbm.at[pl.ds(base, bpw)], dv, isems[0])
        wcp = pltpu.async_copy(w_hbm.at[pl.ds(base, bpw)], wv, isems[1])
        dcp.wait()
        wcp.wait()

        def make_idx(j):
            for kk in range(j * per, (j + 1) * per):
                d = dv[pl.ds(kk * 16, 16)]
                w = wv[pl.ds(kk * 16, 16)]
                d = jnp.minimum(jnp.maximum(d, 0), vd_max)
                w = jnp.minimum(jnp.maximum(w, 0), vw_max)
                cidx[kk // per, pl.ds((kk % per) * 16, 16)] = d * STR + w

        def gather(j, b):
            return pltpu.async_copy(comb_hbm.at[cidx.at[j]], bufs[b], gsems[b])

        def scatter(j, b):
            return pltpu.async_copy(
                bufs[b], out_hbm.at[pl.ds(base + j * C, C), 0], ssems[b])

        gd = [None] * NB
        sd = [None] * NB
        for j in range(min(PRIME, nch)):
            make_idx(j)
            gd[j % NB] = gather(j, j % NB)
        for j in range(min(PRIME, nch), nch):
            make_idx(j)
        for j in range(nch):
            b = j % NB
            gd[b].wait()
            jn = j + PRIME
            if jn < nch:
                bn = jn % NB
                if sd[bn] is not None:
                    sd[bn].wait()
                gd[bn] = gather(jn, bn)
            sd[b] = scatter(j, b)
        for b in range(NB):
            if sd[b] is not None:
                sd[b].wait()

    return k(d_idx, w_idx, comb)


def kernel(depth_indices, width_indices, depth_embed, width_embed):
    seq = depth_indices.shape[0]
    D = depth_embed.shape[1]
    comb = _combine_tables(depth_embed, width_embed)
    d = depth_indices.reshape(seq).astype(jnp.int32)
    w = width_indices.reshape(seq).astype(jnp.int32)
    return _sc_gather(d, w, comb, D,
                      depth_embed.shape[0] - 1, width_embed.shape[0] - 1)
